# Initial kernel scaffold; baseline (speedup 1.0000x reference)
import jax
import jax.numpy as jnp
from jax import lax
from jax.experimental import pallas as pl
from jax.experimental.pallas import tpu as pltpu
from jax.experimental.pallas import tpu_sc as plsc

NC, NS, LANES = 2, 16, 16
NW = NC * NS
CHUNK = 128


def _ceil_to(x, m):
    return -(-x // m) * m


def _pad_edges(src, dst, n_dst, e_pad, w=None):
    e = src.shape[0]
    pad = e_pad - e
    src_p = jnp.concatenate([src, jnp.zeros((pad,), jnp.int32)])
    dst_p = jnp.concatenate([dst, jnp.full((pad,), n_dst, jnp.int32)])
    if w is None:
        return src_p, dst_p, None
    w_p = jnp.concatenate([w, jnp.zeros((pad,), jnp.float32)])
    return src_p, dst_p, w_p


def _stripe_rows(n_rows):
    per = -(-n_rows // NS)
    return _ceil_to(per, 8)


def _seg_sum_call(h, src_p, dst_p, w_p, n_dst, e_pad):
    n_src, d = h.shape
    stripe = _stripe_rows(n_dst + 1)
    acc_rows = stripe * NS
    epw = e_pad // NW
    nchunks = epw // CHUNK
    weighted = w_p is not None

    zeros = jnp.zeros((stripe, d), jnp.float32)
    zeros16 = jnp.zeros((stripe, 16), jnp.float32)
    ones = jnp.ones((CHUNK, 16), jnp.float32)

    mesh = plsc.VectorSubcoreMesh(core_axis_name="c", subcore_axis_name="s")
    scratch = [
        pltpu.VMEM_SHARED((acc_rows, d), jnp.float32),
        pltpu.VMEM_SHARED((acc_rows, 16), jnp.float32),
        pltpu.VMEM((CHUNK, d), jnp.float32),
        pltpu.VMEM((CHUNK,), jnp.int32),
        pltpu.VMEM((CHUNK,), jnp.int32),
        pltpu.VMEM((CHUNK, 16), jnp.float32),
        pltpu.SemaphoreType.DMA,
    ]
    if weighted:
        scratch.append(pltpu.VMEM((CHUNK,), jnp.float32))

    out_type = (jax.ShapeDtypeStruct((NC, acc_rows, d), jnp.float32),
                jax.ShapeDtypeStruct((NC, acc_rows, 16), jnp.float32))

    def body(*refs):
        if weighted:
            (h_r, src_r, dst_r, w_r, z_r, z16_r, ones_r, agg_o, cnt_o,
             acc, cacc, rows, idx_s, idx_d, ones_v, sem, w_v) = refs
        else:
            (h_r, src_r, dst_r, z_r, z16_r, ones_r, agg_o, cnt_o,
             acc, cacc, rows, idx_s, idx_d, ones_v, sem) = refs
            w_r = w_v = None
        c = lax.axis_index("c")
        s = lax.axis_index("s")
        wid = s * NC + c
        sl_stripe = pl.ds(s * stripe, stripe)
        pltpu.sync_copy(z_r, acc.at[sl_stripe])
        pltpu.sync_copy(z16_r, cacc.at[sl_stripe])
        pltpu.sync_copy(ones_r, ones_v)
        plsc.subcore_barrier()

        def chunk(i, carry):
            base = wid * epw + i * CHUNK
            pltpu.sync_copy(src_r.at[pl.ds(base, CHUNK)], idx_s)
            pltpu.sync_copy(dst_r.at[pl.ds(base, CHUNK)], idx_d)
            pltpu.async_copy(h_r.at[idx_s], rows, sem).wait()
            if weighted:
                pltpu.sync_copy(w_r.at[pl.ds(base, CHUNK)], w_v)

                def scale(k, carry2):
                    for u in range(4):
                        r = k * 4 + u
                        wv = w_v[r]
                        for j in range(d // LANES):
                            sl = pl.ds(j * LANES, LANES)
                            rows[r, sl] = rows[r, sl] * wv
                    return carry2

                lax.fori_loop(0, CHUNK // 4, scale, 0)
            pltpu.sync_copy(rows, acc.at[idx_d], add=True)
            pltpu.sync_copy(ones_v, cacc.at[idx_d], add=True)
            return carry

        lax.fori_loop(0, nchunks, chunk, 0)
        plsc.subcore_barrier()
        pltpu.sync_copy(acc.at[sl_stripe], agg_o.at[c].at[sl_stripe])
        pltpu.sync_copy(cacc.at[sl_stripe], cnt_o.at[c].at[sl_stripe])

    fn = pl.kernel(body, out_type, mesh=mesh, scratch_types=scratch)
    if weighted:
        return fn(h, src_p, dst_p, w_p, zeros, zeros16, ones)
    return fn(h, src_p, dst_p, zeros, zeros16, ones)


def _seg_sum_split_call(h4, src_p, dst_p, n_dst, e_pad):
    d4 = 32
    stripe = _stripe_rows(n_dst + 1)
    acc_rows = stripe * NS
    epw = e_pad // NW
    nchunks = epw // CHUNK

    zeros = jnp.zeros((stripe, d4), jnp.float32)
    ones = jnp.ones((CHUNK, d4), jnp.float32)

    mesh = plsc.VectorSubcoreMesh(core_axis_name="c", subcore_axis_name="s")
    scratch = [
        pltpu.VMEM_SHARED((acc_rows, d4), jnp.float32),
        pltpu.VMEM((CHUNK, d4), jnp.float32),
        pltpu.VMEM((CHUNK,), jnp.int32),
        pltpu.VMEM((CHUNK,), jnp.int32),
        pltpu.VMEM((CHUNK,), jnp.int32),
        pltpu.VMEM((CHUNK, d4), jnp.float32),
        pltpu.SemaphoreType.DMA,
    ]
    out_type = jax.ShapeDtypeStruct((5, NC, acc_rows, d4), jnp.float32)

    def body(h_r, src_r, dst_r, z_r, ones_r, out_o,
             acc, rows, idx_s, idx_d, idx4, ones_v, sem):
        c = lax.axis_index("c")
        s = lax.axis_index("s")
        wid = s * NC + c
        sl_stripe = pl.ds(s * stripe, stripe)
        pltpu.sync_copy(ones_r, ones_v)
        for f in range(5):
            pltpu.sync_copy(z_r, acc.at[sl_stripe])
            plsc.subcore_barrier()

            def chunk(i, carry, f=f):
                base = wid * epw + i * CHUNK
                pltpu.sync_copy(dst_r.at[pl.ds(base, CHUNK)], idx_d)
                if f < 4:
                    pltpu.sync_copy(src_r.at[pl.ds(base, CHUNK)], idx_s)

                    def mk(k, carry2):
                        sl = pl.ds(k * LANES, LANES)
                        idx4[sl] = idx_s[sl] * 4 + f
                        return carry2

                    lax.fori_loop(0, CHUNK // LANES, mk, 0)
                    pltpu.async_copy(h_r.at[idx4], rows, sem).wait()
                    pltpu.sync_copy(rows, acc.at[idx_d], add=True)
                else:
                    pltpu.sync_copy(ones_v, acc.at[idx_d], add=True)
                return carry

            lax.fori_loop(0, nchunks, chunk, 0)
            plsc.subcore_barrier()
            pltpu.sync_copy(acc.at[sl_stripe], out_o.at[f].at[c].at[sl_stripe])
            plsc.subcore_barrier()

    fn = pl.kernel(body, out_type, mesh=mesh, scratch_types=scratch)
    return fn(h4, src_p, dst_p, zeros, ones)


def _tc_dense(h_prev, W, b, g, beta, agg_groups, cnt_pieces, blk):
    n, d = h_prev.shape
    grid = n // blk
    b2 = b.reshape(1, d)
    g2 = g.reshape(1, d)
    beta2 = beta.reshape(1, d)
    flat_aggs = [a for grp in agg_groups for a in grp]
    group_sizes = [len(grp) for grp in agg_groups]

    in_specs = [
        pl.BlockSpec((blk, d), lambda i: (i, 0)),
        pl.BlockSpec((d, d), lambda i: (0, 0)),
        pl.BlockSpec((1, d), lambda i: (0, 0)),
        pl.BlockSpec((1, d), lambda i: (0, 0)),
        pl.BlockSpec((1, d), lambda i: (0, 0)),
    ]
    for a in flat_aggs:
        in_specs.append(pl.BlockSpec((blk, a.shape[1]), lambda i: (i, 0)))
    for cp in cnt_pieces:
        in_specs.append(pl.BlockSpec((blk, cp.shape[1]), lambda i: (i, 0)))

    n_aggs = len(flat_aggs)

    def body(hp, Wr, br, gr, betar, *rest):
        out = rest[-1]
        agg_refs = rest[:n_aggs]
        cnt_refs = rest[n_aggs:-1]
        pieces = []
        off = 0
        for sz in group_sizes:
            grp = agg_refs[off:off + sz]
            off += sz
            ssum = grp[0][...]
            for rr in grp[1:]:
                ssum = ssum + rr[...]
            pieces.append(ssum)
        agg = pieces[0] if len(pieces) == 1 else jnp.concatenate(pieces, axis=1)
        cnt = cnt_refs[0][...][:, 0:1]
        for rr in cnt_refs[1:]:
            cnt = cnt + rr[...][:, 0:1]
        deg = jnp.maximum(cnt, 1.0)
        x = jnp.dot(agg, Wr[...], preferred_element_type=jnp.float32) / deg
        x = x + br[...] + hp[...]
        mu = jnp.mean(x, axis=1, keepdims=True)
        var = jnp.mean((x - mu) * (x - mu), axis=1, keepdims=True)
        y = (x - mu) * lax.rsqrt(var + 1e-5) * gr[...] + betar[...]
        out[...] = jnp.maximum(y, 0.0)

    return pl.pallas_call(
        body,
        grid=(grid,),
        in_specs=in_specs,
        out_specs=pl.BlockSpec((blk, d), lambda i: (i, 0)),
        out_shape=jax.ShapeDtypeStruct((n, d), jnp.float32),
    )(h_prev, W, b2, g2, beta2, *flat_aggs, *cnt_pieces)


def kernel(h_pin, h_net, overlap_weights, W_p2n, b_p2n, W_n2n, b_n2n, W_n2p,
           b_n2p, ln1_g, ln1_b, ln2_g, ln2_b, lnp_g, lnp_b,
           src_p2n, dst_p2n, src_n2n, dst_n2n, src_n2p, dst_n2p):
    n_pin, d = h_pin.shape
    n_net = h_net.shape[0]
    grain = NW * CHUNK

    e1p = _ceil_to(src_p2n.shape[0], grain)
    s1, d1, _ = _pad_edges(src_p2n, dst_p2n, n_net, e1p)
    agg1, cnt1 = _seg_sum_call(h_pin, s1, d1, None, n_net, e1p)
    h_net1 = _tc_dense(h_net, W_p2n, b_p2n, ln1_g, ln1_b,
                       [[agg1[0, :n_net], agg1[1, :n_net]]],
                       [cnt1[0, :n_net], cnt1[1, :n_net]], blk=1000)

    e2p = _ceil_to(src_n2n.shape[0], grain)
    s2, d2, w2 = _pad_edges(src_n2n, dst_n2n, n_net, e2p, overlap_weights)
    agg2, cnt2 = _seg_sum_call(h_net1, s2, d2, w2, n_net, e2p)
    h_net2 = _tc_dense(h_net1, W_n2n, b_n2n, ln2_g, ln2_b,
                       [[agg2[0, :n_net], agg2[1, :n_net]]],
                       [cnt2[0, :n_net], cnt2[1, :n_net]], blk=1000)

    e3p = _ceil_to(src_n2p.shape[0], grain)
    s3, d3, _ = _pad_edges(src_n2p, dst_n2p, n_pin, e3p)
    h4 = h_net2.reshape(n_net * 4, 32)
    out3 = _seg_sum_split_call(h4, s3, d3, n_pin, e3p)
    agg_groups = [[out3[f, 0, :n_pin], out3[f, 1, :n_pin]] for f in range(4)]
    cnt_pieces = [out3[4, 0, :n_pin], out3[4, 1, :n_pin]]
    h_pin_out = _tc_dense(h_pin, W_n2p, b_n2p, lnp_g, lnp_b,
                          agg_groups, cnt_pieces, blk=1000)
    return (h_pin_out, h_net2)


# speedup vs baseline: 1.2160x; 1.2160x over previous
import jax
import jax.numpy as jnp
from jax import lax
from jax.experimental import pallas as pl
from jax.experimental.pallas import tpu as pltpu
from jax.experimental.pallas import tpu_sc as plsc

NC, NS, LANES = 2, 16, 16
NW = NC * NS
CHUNK = 128


def _ceil_to(x, m):
    return -(-x // m) * m


def _pad_edges(src, dst, n_dst, e_pad, w=None):
    e = src.shape[0]
    pad = e_pad - e
    src_p = jnp.concatenate([src, jnp.zeros((pad,), jnp.int32)])
    dst_p = jnp.concatenate([dst, jnp.full((pad,), n_dst, jnp.int32)])
    if w is None:
        return src_p, dst_p, None
    w_p = jnp.concatenate([w, jnp.zeros((pad,), jnp.float32)])
    return src_p, dst_p, w_p


def _stripe_rows(n_rows):
    per = -(-n_rows // NS)
    return _ceil_to(per, 32)


def _seg_sum_call(hv, src_p, dst_p, w_p, n_dst, e_pad, npass, width):
    stripe = _stripe_rows(n_dst + 1)
    acc_rows = stripe * NS
    epw = e_pad // NW
    nchunks = epw // CHUNK
    weighted = w_p is not None

    zeros = jnp.zeros((stripe, width), jnp.float32)
    ones = jnp.ones((CHUNK, width), jnp.float32)

    mesh = plsc.VectorSubcoreMesh(core_axis_name="c", subcore_axis_name="s")
    scratch = [
        pltpu.VMEM_SHARED((acc_rows, width), jnp.float32),
        pltpu.VMEM((CHUNK, width), jnp.float32),
        pltpu.VMEM((CHUNK,), jnp.int32),
        pltpu.VMEM((CHUNK,), jnp.int32),
        pltpu.VMEM((CHUNK,), jnp.int32),
        pltpu.VMEM((CHUNK, width), jnp.float32),
        pltpu.SemaphoreType.DMA,
    ]
    if weighted:
        scratch.append(pltpu.VMEM((CHUNK,), jnp.float32))
    out_type = jax.ShapeDtypeStruct((npass + 1, NC, acc_rows, width),
                                    jnp.float32)

    def body(*refs):
        if weighted:
            (h_r, src_r, dst_r, w_r, z_r, ones_r, out_o,
             acc, rows, idx_s, idx_d, idxp, ones_v, sem, w_v) = refs
        else:
            (h_r, src_r, dst_r, z_r, ones_r, out_o,
             acc, rows, idx_s, idx_d, idxp, ones_v, sem) = refs
            w_r = w_v = None
        c = lax.axis_index("c")
        s = lax.axis_index("s")
        wid = s * NC + c
        q = stripe // 4
        pltpu.sync_copy(ones_r, ones_v)
        for f in range(npass + 1):
            for j in range(4):
                pltpu.sync_copy(z_r.at[pl.ds(j * q, q)],
                                acc.at[pl.ds(s * stripe + j * q, q)])
            plsc.subcore_barrier()

            def chunk(i, carry, f=f):
                base = wid * epw + i * CHUNK
                pltpu.sync_copy(dst_r.at[pl.ds(base, CHUNK)], idx_d)
                if f < npass:
                    pltpu.sync_copy(src_r.at[pl.ds(base, CHUNK)], idx_s)

                    def mk(k, carry2):
                        sl = pl.ds(k * LANES, LANES)
                        idxp[sl] = idx_s[sl] * npass + f
                        return carry2

                    lax.fori_loop(0, CHUNK // LANES, mk, 0)
                    pltpu.async_copy(h_r.at[idxp], rows, sem).wait()
                    if weighted:
                        pltpu.sync_copy(w_r.at[pl.ds(base, CHUNK)], w_v)

                        def scale(k, carry2):
                            wv16 = w_v[pl.ds(k * LANES, LANES)]
                            for u in range(LANES):
                                r = k * LANES + u
                                wvu = wv16[u]
                                for j in range(width // LANES):
                                    sl = pl.ds(j * LANES, LANES)
                                    rows[r, sl] = rows[r, sl] * wvu
                            return carry2

                        lax.fori_loop(0, CHUNK // LANES, scale, 0)
                    pltpu.sync_copy(rows, acc.at[idx_d], add=True)
                else:
                    pltpu.sync_copy(ones_v, acc.at[idx_d], add=True)
                return carry

            lax.fori_loop(0, nchunks, chunk, 0)
            plsc.subcore_barrier()
            for j in range(4):
                slj = pl.ds(s * stripe + j * q, q)
                pltpu.sync_copy(acc.at[slj], out_o.at[f].at[c].at[slj])
            plsc.subcore_barrier()

    fn = pl.kernel(body, out_type, mesh=mesh, scratch_types=scratch,
                   compiler_params=pltpu.CompilerParams(
                       use_tc_tiling_on_sc=False))
    if weighted:
        return fn(hv, src_p, dst_p, w_p, zeros, ones)
    return fn(hv, src_p, dst_p, zeros, ones)


def _tc_dense(h_prev, W, b, g, beta, agg_groups, cnt_pieces, blk):
    n, d = h_prev.shape
    grid = n // blk
    b2 = b.reshape(1, d)
    g2 = g.reshape(1, d)
    beta2 = beta.reshape(1, d)
    flat_aggs = [a for grp in agg_groups for a in grp]
    group_sizes = [len(grp) for grp in agg_groups]

    in_specs = [
        pl.BlockSpec((blk, d), lambda i: (i, 0)),
        pl.BlockSpec((d, d), lambda i: (0, 0)),
        pl.BlockSpec((1, d), lambda i: (0, 0)),
        pl.BlockSpec((1, d), lambda i: (0, 0)),
        pl.BlockSpec((1, d), lambda i: (0, 0)),
    ]
    for a in flat_aggs:
        in_specs.append(pl.BlockSpec((blk, a.shape[1]), lambda i: (i, 0)))
    for cp in cnt_pieces:
        in_specs.append(pl.BlockSpec((blk, cp.shape[1]), lambda i: (i, 0)))

    n_aggs = len(flat_aggs)

    def body(hp, Wr, br, gr, betar, *rest):
        out = rest[-1]
        agg_refs = rest[:n_aggs]
        cnt_refs = rest[n_aggs:-1]
        pieces = []
        off = 0
        for sz in group_sizes:
            grp = agg_refs[off:off + sz]
            off += sz
            ssum = grp[0][...]
            for rr in grp[1:]:
                ssum = ssum + rr[...]
            pieces.append(ssum)
        agg = pieces[0] if len(pieces) == 1 else jnp.concatenate(pieces, axis=1)
        cnt = cnt_refs[0][...][:, 0:1]
        for rr in cnt_refs[1:]:
            cnt = cnt + rr[...][:, 0:1]
        deg = jnp.maximum(cnt, 1.0)
        x = jnp.dot(agg, Wr[...], preferred_element_type=jnp.float32) / deg
        x = x + br[...] + hp[...]
        mu = jnp.mean(x, axis=1, keepdims=True)
        var = jnp.mean((x - mu) * (x - mu), axis=1, keepdims=True)
        y = (x - mu) * lax.rsqrt(var + 1e-5) * gr[...] + betar[...]
        out[...] = jnp.maximum(y, 0.0)

    return pl.pallas_call(
        body,
        grid=(grid,),
        in_specs=in_specs,
        out_specs=pl.BlockSpec((blk, d), lambda i: (i, 0)),
        out_shape=jax.ShapeDtypeStruct((n, d), jnp.float32),
    )(h_prev, W, b2, g2, beta2, *flat_aggs, *cnt_pieces)


def _stage(h_src, h_prev, src, dst, w, Wm, b, g, beta, npass, blk):
    n_src, d = h_src.shape
    n_dst = h_prev.shape[0]
    width = d // npass
    e_pad = _ceil_to(src.shape[0], NW * CHUNK)
    sp, dp, wp = _pad_edges(src, dst, n_dst, e_pad, w)
    hv = h_src.reshape(n_src * npass, width)
    out = _seg_sum_call(hv, sp, dp, wp, n_dst, e_pad, npass, width)
    agg_groups = [[out[f, 0, :n_dst], out[f, 1, :n_dst]] for f in range(npass)]
    cnt_pieces = [out[npass, 0, :n_dst], out[npass, 1, :n_dst]]
    return _tc_dense(h_prev, Wm, b, g, beta, agg_groups, cnt_pieces, blk)


def kernel(h_pin, h_net, overlap_weights, W_p2n, b_p2n, W_n2n, b_n2n, W_n2p,
           b_n2p, ln1_g, ln1_b, ln2_g, ln2_b, lnp_g, lnp_b,
           src_p2n, dst_p2n, src_n2n, dst_n2n, src_n2p, dst_n2p):
    h_net1 = _stage(h_pin, h_net, src_p2n, dst_p2n, None,
                    W_p2n, b_p2n, ln1_g, ln1_b, npass=2, blk=1000)
    h_net2 = _stage(h_net1, h_net1, src_n2n, dst_n2n, overlap_weights,
                    W_n2n, b_n2n, ln2_g, ln2_b, npass=2, blk=1000)
    h_pin_out = _stage(h_net2, h_pin, src_n2p, dst_n2p, None,
                       W_n2p, b_n2p, lnp_g, lnp_b, npass=8, blk=1000)
    return (h_pin_out, h_net2)


# speedup vs baseline: 1.8570x; 1.5271x over previous
import jax
import jax.numpy as jnp
from jax import lax
from jax.experimental import pallas as pl
from jax.experimental.pallas import tpu as pltpu
from jax.experimental.pallas import tpu_sc as plsc

NC, NS, LANES = 2, 16, 16
NW = NC * NS
CHUNK = 128


def _ceil_to(x, m):
    return -(-x // m) * m


def _pad_edges(src, dst, n_dst, e_pad, w=None):
    e = src.shape[0]
    pad = e_pad - e
    src_p = jnp.concatenate([src, jnp.zeros((pad,), jnp.int32)])
    dst_p = jnp.concatenate([dst, jnp.full((pad,), n_dst, jnp.int32)])
    if w is None:
        return src_p, dst_p, None
    w_p = jnp.concatenate([w, jnp.zeros((pad,), jnp.float32)])
    return src_p, dst_p, w_p


def _stripe_rows(n_rows):
    per = -(-n_rows // NS)
    return _ceil_to(per, 32)


SUPER = 4
STEP = SUPER * CHUNK


def _seg_sum_call(hv, src_p, dst_p, w_p, n_dst, e_pad, npass, width,
                  merge_cnt):
    stripe = _stripe_rows(n_dst + 1)
    acc_rows = stripe * NS
    epw = e_pad // NW
    nsteps = epw // STEP
    assert nsteps % 2 == 0 and nsteps >= 2
    weighted = w_p is not None
    if not merge_cnt:
        assert width == 16

    zeros = jnp.zeros((stripe, width), jnp.float32)
    zeros16 = jnp.zeros((stripe, 16), jnp.float32)
    ones = jnp.ones((CHUNK, 16), jnp.float32)

    mesh = plsc.VectorSubcoreMesh(core_axis_name="c", subcore_axis_name="s")
    scratch = [
        pltpu.VMEM_SHARED((acc_rows, width), jnp.float32),
        pltpu.VMEM((2, STEP, width), jnp.float32),
        pltpu.VMEM((2, SUPER, CHUNK), jnp.int32),
        pltpu.VMEM((2, SUPER, CHUNK), jnp.int32),
        pltpu.VMEM((2, SUPER, CHUNK), jnp.int32),
        pltpu.VMEM((CHUNK, 16), jnp.float32),
        pltpu.SemaphoreType.DMA,
        pltpu.SemaphoreType.DMA,
        pltpu.SemaphoreType.DMA,
        pltpu.SemaphoreType.DMA,
    ]
    if merge_cnt:
        scratch.append(pltpu.VMEM_SHARED((acc_rows, 16), jnp.float32))
    if weighted:
        scratch.append(pltpu.VMEM((2, SUPER, CHUNK), jnp.float32))
    out_shapes = (jax.ShapeDtypeStruct((npass, NC, acc_rows, width),
                                       jnp.float32),
                  jax.ShapeDtypeStruct((NC, acc_rows, 16), jnp.float32))

    def body(*refs):
        it = iter(refs)
        h_r = next(it)
        src_r = next(it)
        dst_r = next(it)
        w_r = next(it) if weighted else None
        z_r = next(it)
        z16_r = next(it)
        ones_r = next(it)
        agg_o = next(it)
        cnt_o = next(it)
        acc = next(it)
        rows = next(it)
        idx_s = next(it)
        idx_d = next(it)
        idxp = next(it)
        ones_v = next(it)
        semI = (next(it), next(it))
        semG = (next(it), next(it))
        cacc = next(it) if merge_cnt else None
        w_v = next(it) if weighted else None

        c = lax.axis_index("c")
        s = lax.axis_index("s")
        wid = s * NC + c
        q = stripe // 4
        pltpu.sync_copy(ones_r, ones_v)

        total_passes = npass if merge_cnt else npass + 1
        for f in range(total_passes):
            gather_pass = f < npass
            do_cnt = (merge_cnt and f == 0) or not gather_pass
            cnt_acc = cacc if merge_cnt else acc

            for j in range(4):
                pltpu.sync_copy(z_r.at[pl.ds(j * q, q)],
                                acc.at[pl.ds(s * stripe + j * q, q)])
            if merge_cnt and f == 0:
                pltpu.sync_copy(z16_r, cacc.at[pl.ds(s * stripe, stripe)])
            plsc.subcore_barrier()

            def fire_idx(par, i):
                base = wid * epw + i * STEP
                for j in range(SUPER):
                    bj = base + j * CHUNK
                    pltpu.async_copy(dst_r.at[pl.ds(bj, CHUNK)],
                                     idx_d.at[par].at[j], semI[par])
                    if gather_pass:
                        pltpu.async_copy(src_r.at[pl.ds(bj, CHUNK)],
                                         idx_s.at[par].at[j], semI[par])
                        if weighted:
                            pltpu.async_copy(w_r.at[pl.ds(bj, CHUNK)],
                                             w_v.at[par].at[j], semI[par])

            def wait_idx(par):
                for j in range(SUPER):
                    pltpu.make_async_copy(dst_r.at[pl.ds(0, CHUNK)],
                                          idx_d.at[par].at[j],
                                          semI[par]).wait()
                    if gather_pass:
                        pltpu.make_async_copy(src_r.at[pl.ds(0, CHUNK)],
                                              idx_s.at[par].at[j],
                                              semI[par]).wait()
                        if weighted:
                            pltpu.make_async_copy(w_r.at[pl.ds(0, CHUNK)],
                                                  w_v.at[par].at[j],
                                                  semI[par]).wait()

            def mk_idx(par, f=f):
                for j in range(SUPER):
                    def mk(k, carry2, j=j):
                        sl = pl.ds(k * LANES, LANES)
                        idxp[par, j, sl] = idx_s[par, j, sl] * npass + f
                        return carry2
                    lax.fori_loop(0, CHUNK // LANES, mk, 0)

            def fire_gather(par):
                for j in range(SUPER):
                    pltpu.async_copy(h_r.at[idxp.at[par].at[j]],
                                     rows.at[par].at[pl.ds(j * CHUNK, CHUNK)],
                                     semG[par])

            def wait_gather(par):
                for j in range(SUPER):
                    pltpu.make_async_copy(
                        h_r.at[pl.ds(0, CHUNK)],
                        rows.at[par].at[pl.ds(j * CHUNK, CHUNK)],
                        semG[par]).wait()

            def scale_rows(par):
                for j in range(SUPER):
                    def scale(k, carry2, j=j):
                        wv16 = w_v[par, j, pl.ds(k * LANES, LANES)]
                        for u in range(LANES):
                            r = j * CHUNK + k * LANES + u
                            wvu = wv16[u]
                            for t in range(width // LANES):
                                sl = pl.ds(t * LANES, LANES)
                                rows[par, r, sl] = rows[par, r, sl] * wvu
                        return carry2
                    lax.fori_loop(0, CHUNK // LANES, scale, 0)

            def do_scatters(par):
                for j in range(SUPER):
                    if gather_pass:
                        pltpu.sync_copy(
                            rows.at[par].at[pl.ds(j * CHUNK, CHUNK)],
                            acc.at[idx_d.at[par].at[j]], add=True)
                    if do_cnt:
                        pltpu.sync_copy(ones_v,
                                        cnt_acc.at[idx_d.at[par].at[j]],
                                        add=True)

            fire_idx(0, 0)
            fire_idx(1, 1)
            wait_idx(0)
            if gather_pass:
                mk_idx(0)
                fire_gather(0)

            def pair(g, carry):
                for par in (0, 1):
                    i = g * 2 + par

                    @pl.when(i + 1 < nsteps)
                    def _():
                        wait_idx(1 - par)
                        if gather_pass:
                            mk_idx(1 - par)
                            fire_gather(1 - par)

                    if gather_pass:
                        wait_gather(par)
                        if weighted:
                            scale_rows(par)
                    do_scatters(par)

                    @pl.when(i + 2 < nsteps)
                    def _():
                        fire_idx(par, i + 2)
                return carry

            lax.fori_loop(0, nsteps // 2, pair, 0)
            plsc.subcore_barrier()
            if gather_pass:
                for j in range(4):
                    slj = pl.ds(s * stripe + j * q, q)
                    pltpu.sync_copy(acc.at[slj], agg_o.at[f].at[c].at[slj])
            if do_cnt:
                for j in range(4):
                    slj = pl.ds(s * stripe + j * q, q)
                    pltpu.sync_copy(cnt_acc.at[slj], cnt_o.at[c].at[slj])
            plsc.subcore_barrier()

    fn = pl.kernel(body, out_shapes, mesh=mesh, scratch_types=scratch,
                   compiler_params=pltpu.CompilerParams(
                       use_tc_tiling_on_sc=False))
    if weighted:
        return fn(hv, src_p, dst_p, w_p, zeros, zeros16, ones)
    return fn(hv, src_p, dst_p, zeros, zeros16, ones)


def _tc_dense(h_prev, W, b, g, beta, agg_groups, cnt_pieces, blk):
    n, d = h_prev.shape
    grid = n // blk
    b2 = b.reshape(1, d)
    g2 = g.reshape(1, d)
    beta2 = beta.reshape(1, d)
    flat_aggs = [a for grp in agg_groups for a in grp]
    group_sizes = [len(grp) for grp in agg_groups]

    in_specs = [
        pl.BlockSpec((blk, d), lambda i: (i, 0)),
        pl.BlockSpec((d, d), lambda i: (0, 0)),
        pl.BlockSpec((1, d), lambda i: (0, 0)),
        pl.BlockSpec((1, d), lambda i: (0, 0)),
        pl.BlockSpec((1, d), lambda i: (0, 0)),
    ]
    for a in flat_aggs:
        in_specs.append(pl.BlockSpec((blk, a.shape[1]), lambda i: (i, 0)))
    for cp in cnt_pieces:
        in_specs.append(pl.BlockSpec((blk, cp.shape[1]), lambda i: (i, 0)))

    n_aggs = len(flat_aggs)

    def body(hp, Wr, br, gr, betar, *rest):
        out = rest[-1]
        agg_refs = rest[:n_aggs]
        cnt_refs = rest[n_aggs:-1]
        pieces = []
        off = 0
        for sz in group_sizes:
            grp = agg_refs[off:off + sz]
            off += sz
            ssum = grp[0][...]
            for rr in grp[1:]:
                ssum = ssum + rr[...]
            pieces.append(ssum)
        agg = pieces[0] if len(pieces) == 1 else jnp.concatenate(pieces, axis=1)
        cnt = cnt_refs[0][...][:, 0:1]
        for rr in cnt_refs[1:]:
            cnt = cnt + rr[...][:, 0:1]
        deg = jnp.maximum(cnt, 1.0)
        x = jnp.dot(agg, Wr[...], preferred_element_type=jnp.float32) / deg
        x = x + br[...] + hp[...]
        mu = jnp.mean(x, axis=1, keepdims=True)
        var = jnp.mean((x - mu) * (x - mu), axis=1, keepdims=True)
        y = (x - mu) * lax.rsqrt(var + 1e-5) * gr[...] + betar[...]
        out[...] = jnp.maximum(y, 0.0)

    return pl.pallas_call(
        body,
        grid=(grid,),
        in_specs=in_specs,
        out_specs=pl.BlockSpec((blk, d), lambda i: (i, 0)),
        out_shape=jax.ShapeDtypeStruct((n, d), jnp.float32),
    )(h_prev, W, b2, g2, beta2, *flat_aggs, *cnt_pieces)


def _stage(h_src, h_prev, src, dst, w, Wm, b, g, beta, npass, blk):
    n_src, d = h_src.shape
    n_dst = h_prev.shape[0]
    width = d // npass
    e_pad = _ceil_to(src.shape[0], NW * STEP)
    if (e_pad // (NW * STEP)) % 2:
        e_pad += NW * STEP
    sp, dp, wp = _pad_edges(src, dst, n_dst, e_pad, w)
    hv = h_src.reshape(n_src * npass, width)
    agg, cnt = _seg_sum_call(hv, sp, dp, wp, n_dst, e_pad, npass, width,
                             merge_cnt=(width > 16))
    agg_groups = [[agg[f, 0, :n_dst], agg[f, 1, :n_dst]] for f in range(npass)]
    cnt_pieces = [cnt[0, :n_dst], cnt[1, :n_dst]]
    return _tc_dense(h_prev, Wm, b, g, beta, agg_groups, cnt_pieces, blk)


def kernel(h_pin, h_net, overlap_weights, W_p2n, b_p2n, W_n2n, b_n2n, W_n2p,
           b_n2p, ln1_g, ln1_b, ln2_g, ln2_b, lnp_g, lnp_b,
           src_p2n, dst_p2n, src_n2n, dst_n2n, src_n2p, dst_n2p):
    h_net1 = _stage(h_pin, h_net, src_p2n, dst_p2n, None,
                    W_p2n, b_p2n, ln1_g, ln1_b, npass=2, blk=1000)
    h_net2 = _stage(h_net1, h_net1, src_n2n, dst_n2n, overlap_weights,
                    W_n2n, b_n2n, ln2_g, ln2_b, npass=2, blk=1000)
    h_pin_out = _stage(h_net2, h_pin, src_n2p, dst_n2p, None,
                       W_n2p, b_n2p, lnp_g, lnp_b, npass=8, blk=1000)
    return (h_pin_out, h_net2)


# speedup vs baseline: 2.1922x; 1.1805x over previous
import jax
import jax.numpy as jnp
from jax import lax
from jax.experimental import pallas as pl
from jax.experimental.pallas import tpu as pltpu
from jax.experimental.pallas import tpu_sc as plsc

NC, NS, LANES = 2, 16, 16
NW = NC * NS
CHUNK = 128


def _ceil_to(x, m):
    return -(-x // m) * m


def _pad_edges(src, dst, n_dst, e_pad, w=None):
    e = src.shape[0]
    pad = e_pad - e
    src_p = jnp.concatenate([src, jnp.zeros((pad,), jnp.int32)])
    dst_p = jnp.concatenate([dst, jnp.full((pad,), n_dst, jnp.int32)])
    if w is None:
        return src_p, dst_p, None
    w_p = jnp.concatenate([w, jnp.zeros((pad,), jnp.float32)])
    return src_p, dst_p, w_p


def _stripe_rows(n_rows):
    per = -(-n_rows // NS)
    return _ceil_to(per, 32)


SUPER = 4
STEP = SUPER * CHUNK


def _seg_sum_call(hv, src_p, dst_p, w_p, n_dst, e_pad, npass, width,
                  merge_cnt):
    stripe = _stripe_rows(n_dst + 1)
    acc_rows = stripe * NS
    epw = e_pad // NW
    nsteps = epw // STEP
    assert nsteps % 2 == 0 and nsteps >= 2
    weighted = w_p is not None
    if not merge_cnt:
        assert width == 16

    zeros = jnp.zeros((stripe, width), jnp.float32)
    zeros16 = jnp.zeros((stripe, 16), jnp.float32)
    ones = jnp.ones((CHUNK, 16), jnp.float32)

    mesh = plsc.VectorSubcoreMesh(core_axis_name="c", subcore_axis_name="s")
    scratch = [
        pltpu.VMEM_SHARED((acc_rows, width), jnp.float32),
        pltpu.VMEM((2, STEP, width), jnp.float32),
        pltpu.VMEM((2, SUPER, CHUNK), jnp.int32),
        pltpu.VMEM((2, SUPER, CHUNK), jnp.int32),
        pltpu.VMEM((2, SUPER, CHUNK), jnp.int32),
        pltpu.VMEM((2, SUPER, CHUNK), jnp.int32),
        pltpu.VMEM((CHUNK, 16), jnp.float32),
        pltpu.SemaphoreType.DMA,
        pltpu.SemaphoreType.DMA,
        pltpu.SemaphoreType.DMA,
        pltpu.SemaphoreType.DMA,
        pltpu.SemaphoreType.DMA,
        pltpu.SemaphoreType.DMA,
    ]
    if merge_cnt:
        scratch.append(pltpu.VMEM_SHARED((acc_rows, 16), jnp.float32))
    if weighted:
        scratch.append(pltpu.VMEM((2, SUPER, CHUNK), jnp.float32))
    out_shapes = (jax.ShapeDtypeStruct((npass, NC, acc_rows, width),
                                       jnp.float32),
                  jax.ShapeDtypeStruct((NC, acc_rows, 16), jnp.float32))

    def body(*refs):
        it = iter(refs)
        h_r = next(it)
        src_r = next(it)
        dst_r = next(it)
        w_r = next(it) if weighted else None
        z_r = next(it)
        z16_r = next(it)
        ones_r = next(it)
        agg_o = next(it)
        cnt_o = next(it)
        acc = next(it)
        rows = next(it)
        idx_s = next(it)
        idx_d = next(it)
        idxd2 = next(it)
        idxp = next(it)
        ones_v = next(it)
        semI = (next(it), next(it))
        semG = (next(it), next(it))
        semS = (next(it), next(it))
        cacc = next(it) if merge_cnt else None
        w_v = next(it) if weighted else None

        c = lax.axis_index("c")
        s = lax.axis_index("s")
        wid = s * NC + c
        q = stripe // 4
        pltpu.sync_copy(ones_r, ones_v)

        total_passes = npass if merge_cnt else npass + 1
        for f in range(total_passes):
            gather_pass = f < npass
            do_cnt = (merge_cnt and f == 0) or not gather_pass
            cnt_acc = cacc if merge_cnt else acc

            for j in range(4):
                pltpu.sync_copy(z_r.at[pl.ds(j * q, q)],
                                acc.at[pl.ds(s * stripe + j * q, q)])
            if merge_cnt and f == 0:
                pltpu.sync_copy(z16_r, cacc.at[pl.ds(s * stripe, stripe)])
            plsc.subcore_barrier()

            def fire_idx(par, i):
                base = wid * epw + i * STEP
                for j in range(SUPER):
                    bj = base + j * CHUNK
                    pltpu.async_copy(dst_r.at[pl.ds(bj, CHUNK)],
                                     idx_d.at[par].at[j], semI[par])
                    if gather_pass:
                        pltpu.async_copy(src_r.at[pl.ds(bj, CHUNK)],
                                         idx_s.at[par].at[j], semI[par])
                        if weighted:
                            pltpu.async_copy(w_r.at[pl.ds(bj, CHUNK)],
                                             w_v.at[par].at[j], semI[par])

            def wait_idx(par):
                for j in range(SUPER):
                    pltpu.make_async_copy(dst_r.at[pl.ds(0, CHUNK)],
                                          idx_d.at[par].at[j],
                                          semI[par]).wait()
                    if gather_pass:
                        pltpu.make_async_copy(src_r.at[pl.ds(0, CHUNK)],
                                              idx_s.at[par].at[j],
                                              semI[par]).wait()
                        if weighted:
                            pltpu.make_async_copy(w_r.at[pl.ds(0, CHUNK)],
                                                  w_v.at[par].at[j],
                                                  semI[par]).wait()

            def mk_idx(par, f=f):
                for j in range(SUPER):
                    def mk(k, carry2, j=j):
                        sl = pl.ds(k * LANES, LANES)
                        idxp[par, j, sl] = idx_s[par, j, sl] * npass + f
                        return carry2
                    lax.fori_loop(0, CHUNK // LANES, mk, 0)

            def fire_gather(par):
                for j in range(SUPER):
                    pltpu.async_copy(h_r.at[idxp.at[par].at[j]],
                                     rows.at[par].at[pl.ds(j * CHUNK, CHUNK)],
                                     semG[par])

            def wait_gather(par):
                for j in range(SUPER):
                    pltpu.make_async_copy(
                        h_r.at[pl.ds(0, CHUNK)],
                        rows.at[par].at[pl.ds(j * CHUNK, CHUNK)],
                        semG[par]).wait()

            def scale_rows(par):
                for j in range(SUPER):
                    def scale(k, carry2, j=j):
                        wv16 = w_v[par, j, pl.ds(k * LANES, LANES)]
                        for u in range(LANES):
                            r = j * CHUNK + k * LANES + u
                            wvu = wv16[u]
                            for t in range(width // LANES):
                                sl = pl.ds(t * LANES, LANES)
                                rows[par, r, sl] = rows[par, r, sl] * wvu
                        return carry2
                    lax.fori_loop(0, CHUNK // LANES, scale, 0)

            def cp_idxd(par):
                for j in range(SUPER):
                    def cp(k, carry2, j=j):
                        sl = pl.ds(k * LANES, LANES)
                        idxd2[par, j, sl] = idx_d[par, j, sl]
                        return carry2
                    lax.fori_loop(0, CHUNK // LANES, cp, 0)

            def fire_scatters(par):
                for j in range(SUPER):
                    if gather_pass:
                        pltpu.async_copy(
                            rows.at[par].at[pl.ds(j * CHUNK, CHUNK)],
                            acc.at[idxd2.at[par].at[j]], semS[par], add=True)
                    if do_cnt:
                        pltpu.async_copy(ones_v,
                                         cnt_acc.at[idxd2.at[par].at[j]],
                                         semS[par], add=True)

            def drain_scatters(par):
                for j in range(SUPER):
                    if gather_pass:
                        pltpu.make_async_copy(
                            rows.at[par].at[pl.ds(j * CHUNK, CHUNK)],
                            acc.at[pl.ds(0, CHUNK)], semS[par]).wait()
                    if do_cnt:
                        pltpu.make_async_copy(
                            ones_v, cnt_acc.at[pl.ds(0, CHUNK)],
                            semS[par]).wait()

            fire_idx(0, 0)
            fire_idx(1, 1)
            wait_idx(0)
            cp_idxd(0)
            if gather_pass:
                mk_idx(0)
                fire_gather(0)

            def pair(g, carry):
                for par in (0, 1):
                    i = g * 2 + par

                    if gather_pass:
                        wait_gather(par)
                        if weighted:
                            scale_rows(par)
                    fire_scatters(par)

                    @pl.when(i + 2 < nsteps)
                    def _():
                        fire_idx(par, i + 2)

                    @pl.when(i >= 1)
                    def _():
                        drain_scatters(1 - par)

                    @pl.when(i + 1 < nsteps)
                    def _():
                        wait_idx(1 - par)
                        cp_idxd(1 - par)
                        if gather_pass:
                            mk_idx(1 - par)
                            fire_gather(1 - par)
                return carry

            lax.fori_loop(0, nsteps // 2, pair, 0)
            drain_scatters(1)
            plsc.subcore_barrier()
            if gather_pass:
                for j in range(4):
                    slj = pl.ds(s * stripe + j * q, q)
                    pltpu.sync_copy(acc.at[slj], agg_o.at[f].at[c].at[slj])
            if do_cnt:
                for j in range(4):
                    slj = pl.ds(s * stripe + j * q, q)
                    pltpu.sync_copy(cnt_acc.at[slj], cnt_o.at[c].at[slj])
            plsc.subcore_barrier()

    fn = pl.kernel(body, out_shapes, mesh=mesh, scratch_types=scratch,
                   compiler_params=pltpu.CompilerParams(
                       use_tc_tiling_on_sc=False))
    if weighted:
        return fn(hv, src_p, dst_p, w_p, zeros, zeros16, ones)
    return fn(hv, src_p, dst_p, zeros, zeros16, ones)


def _tc_dense(h_prev, W, b, g, beta, agg, cnt, npass, blk):
    n, d = h_prev.shape
    width = agg.shape[3]
    grid = n // blk
    b2 = b.reshape(1, d)
    g2 = g.reshape(1, d)
    beta2 = beta.reshape(1, d)

    in_specs = [
        pl.BlockSpec((blk, d), lambda i: (i, 0)),
        pl.BlockSpec((d, d), lambda i: (0, 0)),
        pl.BlockSpec((1, d), lambda i: (0, 0)),
        pl.BlockSpec((1, d), lambda i: (0, 0)),
        pl.BlockSpec((1, d), lambda i: (0, 0)),
    ]
    operands = [h_prev, W, b2, g2, beta2]
    for f in range(npass):
        for cc in range(NC):
            in_specs.append(pl.BlockSpec(
                (1, 1, blk, width), lambda i, f=f, cc=cc: (f, cc, i, 0)))
            operands.append(agg)
    for cc in range(NC):
        in_specs.append(pl.BlockSpec(
            (1, blk, 16), lambda i, cc=cc: (cc, i, 0)))
        operands.append(cnt)

    def body(hp, Wr, br, gr, betar, *rest):
        out = rest[-1]
        agg_refs = rest[:npass * NC]
        cnt_refs = rest[npass * NC:-1]
        pieces = []
        for f in range(npass):
            ssum = agg_refs[f * NC][0, 0]
            for cc in range(1, NC):
                ssum = ssum + agg_refs[f * NC + cc][0, 0]
            pieces.append(ssum)
        aggb = pieces[0] if npass == 1 else jnp.concatenate(pieces, axis=1)
        cntb = cnt_refs[0][0][:, 0:1]
        for rr in cnt_refs[1:]:
            cntb = cntb + rr[0][:, 0:1]
        deg = jnp.maximum(cntb, 1.0)
        x = jnp.dot(aggb, Wr[...], preferred_element_type=jnp.float32) / deg
        x = x + br[...] + hp[...]
        mu = jnp.mean(x, axis=1, keepdims=True)
        var = jnp.mean((x - mu) * (x - mu), axis=1, keepdims=True)
        y = (x - mu) * lax.rsqrt(var + 1e-5) * gr[...] + betar[...]
        out[...] = jnp.maximum(y, 0.0)

    return pl.pallas_call(
        body,
        grid=(grid,),
        in_specs=in_specs,
        out_specs=pl.BlockSpec((blk, d), lambda i: (i, 0)),
        out_shape=jax.ShapeDtypeStruct((n, d), jnp.float32),
    )(*operands)


def _stage(h_src, h_prev, src, dst, w, Wm, b, g, beta, npass, blk):
    n_src, d = h_src.shape
    n_dst = h_prev.shape[0]
    width = d // npass
    e_pad = _ceil_to(src.shape[0], NW * STEP)
    if (e_pad // (NW * STEP)) % 2:
        e_pad += NW * STEP
    sp, dp, wp = _pad_edges(src, dst, n_dst, e_pad, w)
    hv = h_src.reshape(n_src * npass, width)
    agg, cnt = _seg_sum_call(hv, sp, dp, wp, n_dst, e_pad, npass, width,
                             merge_cnt=(width > 16))
    return _tc_dense(h_prev, Wm, b, g, beta, agg, cnt, npass, blk)


def kernel(h_pin, h_net, overlap_weights, W_p2n, b_p2n, W_n2n, b_n2n, W_n2p,
           b_n2p, ln1_g, ln1_b, ln2_g, ln2_b, lnp_g, lnp_b,
           src_p2n, dst_p2n, src_n2n, dst_n2n, src_n2p, dst_n2p):
    h_net1 = _stage(h_pin, h_net, src_p2n, dst_p2n, None,
                    W_p2n, b_p2n, ln1_g, ln1_b, npass=2, blk=1000)
    h_net2 = _stage(h_net1, h_net1, src_n2n, dst_n2n, overlap_weights,
                    W_n2n, b_n2n, ln2_g, ln2_b, npass=2, blk=1000)
    h_pin_out = _stage(h_net2, h_pin, src_n2p, dst_n2p, None,
                       W_n2p, b_n2p, lnp_g, lnp_b, npass=8, blk=1000)
    return (h_pin_out, h_net2)


# speedup vs baseline: 2.2126x; 1.0093x over previous
import jax
import jax.numpy as jnp
from jax import lax
from jax.experimental import pallas as pl
from jax.experimental.pallas import tpu as pltpu
from jax.experimental.pallas import tpu_sc as plsc

NC, NS, LANES = 2, 16, 16
NW = NC * NS
CHUNK = 128


def _ceil_to(x, m):
    return -(-x // m) * m


def _pad_edges(src, dst, n_dst, e_pad, w=None):
    e = src.shape[0]
    pad = e_pad - e
    src_p = jnp.concatenate([src, jnp.zeros((pad,), jnp.int32)])
    dst_p = jnp.concatenate([dst, jnp.full((pad,), n_dst, jnp.int32)])
    if w is None:
        return src_p, dst_p, None
    w_p = jnp.concatenate([w, jnp.zeros((pad,), jnp.float32)])
    return src_p, dst_p, w_p


def _stripe_rows(n_rows):
    per = -(-n_rows // NS)
    return _ceil_to(per, 32)


def _seg_sum_call(hv, src_p, dst_p, w_p, n_dst, e_pad, npass, width,
                  merge_cnt, super_):
    SUPER = super_
    STEP = SUPER * CHUNK
    stripe = _stripe_rows(n_dst + 1)
    acc_rows = stripe * NS
    epw = e_pad // NW
    nsteps = epw // STEP
    assert nsteps % 2 == 0 and nsteps >= 2
    weighted = w_p is not None
    if not merge_cnt:
        assert width == 16

    zeros = jnp.zeros((stripe, width), jnp.float32)
    zeros16 = jnp.zeros((stripe, 16), jnp.float32)
    ones = jnp.ones((CHUNK, 16), jnp.float32)

    mesh = plsc.VectorSubcoreMesh(core_axis_name="c", subcore_axis_name="s")
    scratch = [
        pltpu.VMEM_SHARED((acc_rows, width), jnp.float32),
        pltpu.VMEM((2, STEP, width), jnp.float32),
        pltpu.VMEM((2, SUPER, CHUNK), jnp.int32),
        pltpu.VMEM((2, SUPER, CHUNK), jnp.int32),
        pltpu.VMEM((2, SUPER, CHUNK), jnp.int32),
        pltpu.VMEM((2, SUPER, CHUNK), jnp.int32),
        pltpu.VMEM((CHUNK, 16), jnp.float32),
        pltpu.SemaphoreType.DMA,
        pltpu.SemaphoreType.DMA,
        pltpu.SemaphoreType.DMA,
        pltpu.SemaphoreType.DMA,
        pltpu.SemaphoreType.DMA,
        pltpu.SemaphoreType.DMA,
    ]
    if merge_cnt:
        scratch.append(pltpu.VMEM_SHARED((acc_rows, 16), jnp.float32))
    if weighted:
        scratch.append(pltpu.VMEM((2, SUPER, CHUNK), jnp.float32))
    out_shapes = (jax.ShapeDtypeStruct((npass, NC, acc_rows, width),
                                       jnp.float32),
                  jax.ShapeDtypeStruct((NC, acc_rows, 16), jnp.float32))

    def body(*refs):
        it = iter(refs)
        h_r = next(it)
        src_r = next(it)
        dst_r = next(it)
        w_r = next(it) if weighted else None
        z_r = next(it)
        z16_r = next(it)
        ones_r = next(it)
        agg_o = next(it)
        cnt_o = next(it)
        acc = next(it)
        rows = next(it)
        idx_s = next(it)
        idx_d = next(it)
        idxd2 = next(it)
        idxp = next(it)
        ones_v = next(it)
        semI = (next(it), next(it))
        semG = (next(it), next(it))
        semS = (next(it), next(it))
        cacc = next(it) if merge_cnt else None
        w_v = next(it) if weighted else None

        c = lax.axis_index("c")
        s = lax.axis_index("s")
        wid = s * NC + c
        q = stripe // 4
        pltpu.sync_copy(ones_r, ones_v)

        total_passes = npass if merge_cnt else npass + 1
        for f in range(total_passes):
            gather_pass = f < npass
            do_cnt = (merge_cnt and f == 0) or not gather_pass
            cnt_acc = cacc if merge_cnt else acc

            for j in range(4):
                pltpu.sync_copy(z_r.at[pl.ds(j * q, q)],
                                acc.at[pl.ds(s * stripe + j * q, q)])
            if merge_cnt and f == 0:
                pltpu.sync_copy(z16_r, cacc.at[pl.ds(s * stripe, stripe)])
            plsc.subcore_barrier()

            def fire_idx(par, i):
                base = wid * epw + i * STEP
                for j in range(SUPER):
                    bj = base + j * CHUNK
                    pltpu.async_copy(dst_r.at[pl.ds(bj, CHUNK)],
                                     idx_d.at[par].at[j], semI[par])
                    if gather_pass:
                        pltpu.async_copy(src_r.at[pl.ds(bj, CHUNK)],
                                         idx_s.at[par].at[j], semI[par])
                        if weighted:
                            pltpu.async_copy(w_r.at[pl.ds(bj, CHUNK)],
                                             w_v.at[par].at[j], semI[par])

            def wait_idx(par):
                for j in range(SUPER):
                    pltpu.make_async_copy(dst_r.at[pl.ds(0, CHUNK)],
                                          idx_d.at[par].at[j],
                                          semI[par]).wait()
                    if gather_pass:
                        pltpu.make_async_copy(src_r.at[pl.ds(0, CHUNK)],
                                              idx_s.at[par].at[j],
                                              semI[par]).wait()
                        if weighted:
                            pltpu.make_async_copy(w_r.at[pl.ds(0, CHUNK)],
                                                  w_v.at[par].at[j],
                                                  semI[par]).wait()

            def mk_idx(par, f=f):
                for j in range(SUPER):
                    def mk(k, carry2, j=j):
                        sl = pl.ds(k * LANES, LANES)
                        idxp[par, j, sl] = idx_s[par, j, sl] * npass + f
                        return carry2
                    lax.fori_loop(0, CHUNK // LANES, mk, 0)

            def fire_gather(par):
                for j in range(SUPER):
                    pltpu.async_copy(h_r.at[idxp.at[par].at[j]],
                                     rows.at[par].at[pl.ds(j * CHUNK, CHUNK)],
                                     semG[par])

            def wait_gather(par):
                for j in range(SUPER):
                    pltpu.make_async_copy(
                        h_r.at[pl.ds(0, CHUNK)],
                        rows.at[par].at[pl.ds(j * CHUNK, CHUNK)],
                        semG[par]).wait()

            def scale_rows(par):
                for j in range(SUPER):
                    def scale(k, carry2, j=j):
                        wv16 = w_v[par, j, pl.ds(k * LANES, LANES)]
                        for u in range(LANES):
                            r = j * CHUNK + k * LANES + u
                            wvu = wv16[u]
                            for t in range(width // LANES):
                                sl = pl.ds(t * LANES, LANES)
                                rows[par, r, sl] = rows[par, r, sl] * wvu
                        return carry2
                    lax.fori_loop(0, CHUNK // LANES, scale, 0)

            def cp_idxd(par):
                for j in range(SUPER):
                    def cp(k, carry2, j=j):
                        sl = pl.ds(k * LANES, LANES)
                        idxd2[par, j, sl] = idx_d[par, j, sl]
                        return carry2
                    lax.fori_loop(0, CHUNK // LANES, cp, 0)

            def fire_scatters(par):
                for j in range(SUPER):
                    if gather_pass:
                        pltpu.async_copy(
                            rows.at[par].at[pl.ds(j * CHUNK, CHUNK)],
                            acc.at[idxd2.at[par].at[j]], semS[par], add=True)
                    if do_cnt:
                        pltpu.async_copy(ones_v,
                                         cnt_acc.at[idxd2.at[par].at[j]],
                                         semS[par], add=True)

            def drain_scatters(par):
                for j in range(SUPER):
                    if gather_pass:
                        pltpu.make_async_copy(
                            rows.at[par].at[pl.ds(j * CHUNK, CHUNK)],
                            acc.at[pl.ds(0, CHUNK)], semS[par]).wait()
                    if do_cnt:
                        pltpu.make_async_copy(
                            ones_v, cnt_acc.at[pl.ds(0, CHUNK)],
                            semS[par]).wait()

            fire_idx(0, 0)
            fire_idx(1, 1)
            wait_idx(0)
            cp_idxd(0)
            if gather_pass:
                mk_idx(0)
                fire_gather(0)

            def pair(g, carry):
                for par in (0, 1):
                    i = g * 2 + par

                    if gather_pass:
                        wait_gather(par)
                        if weighted:
                            scale_rows(par)
                    fire_scatters(par)

                    @pl.when(i + 2 < nsteps)
                    def _():
                        fire_idx(par, i + 2)

                    @pl.when(i >= 1)
                    def _():
                        drain_scatters(1 - par)

                    @pl.when(i + 1 < nsteps)
                    def _():
                        wait_idx(1 - par)
                        cp_idxd(1 - par)
                        if gather_pass:
                            mk_idx(1 - par)
                            fire_gather(1 - par)
                return carry

            lax.fori_loop(0, nsteps // 2, pair, 0)
            drain_scatters(1)
            plsc.subcore_barrier()
            if gather_pass:
                for j in range(4):
                    slj = pl.ds(s * stripe + j * q, q)
                    pltpu.sync_copy(acc.at[slj], agg_o.at[f].at[c].at[slj])
            if do_cnt:
                for j in range(4):
                    slj = pl.ds(s * stripe + j * q, q)
                    pltpu.sync_copy(cnt_acc.at[slj], cnt_o.at[c].at[slj])
            plsc.subcore_barrier()

    fn = pl.kernel(body, out_shapes, mesh=mesh, scratch_types=scratch,
                   compiler_params=pltpu.CompilerParams(
                       use_tc_tiling_on_sc=False))
    if weighted:
        return fn(hv, src_p, dst_p, w_p, zeros, zeros16, ones)
    return fn(hv, src_p, dst_p, zeros, zeros16, ones)


def _tc_dense(h_prev, W, b, g, beta, agg, cnt, npass, blk):
    n, d = h_prev.shape
    width = agg.shape[3]
    grid = n // blk
    b2 = b.reshape(1, d)
    g2 = g.reshape(1, d)
    beta2 = beta.reshape(1, d)

    in_specs = [
        pl.BlockSpec((blk, d), lambda i: (i, 0)),
        pl.BlockSpec((d, d), lambda i: (0, 0)),
        pl.BlockSpec((1, d), lambda i: (0, 0)),
        pl.BlockSpec((1, d), lambda i: (0, 0)),
        pl.BlockSpec((1, d), lambda i: (0, 0)),
    ]
    operands = [h_prev, W, b2, g2, beta2]
    for f in range(npass):
        for cc in range(NC):
            in_specs.append(pl.BlockSpec(
                (1, 1, blk, width), lambda i, f=f, cc=cc: (f, cc, i, 0)))
            operands.append(agg)
    for cc in range(NC):
        in_specs.append(pl.BlockSpec(
            (1, blk, 16), lambda i, cc=cc: (cc, i, 0)))
        operands.append(cnt)

    def body(hp, Wr, br, gr, betar, *rest):
        out = rest[-1]
        agg_refs = rest[:npass * NC]
        cnt_refs = rest[npass * NC:-1]
        pieces = []
        for f in range(npass):
            ssum = agg_refs[f * NC][0, 0]
            for cc in range(1, NC):
                ssum = ssum + agg_refs[f * NC + cc][0, 0]
            pieces.append(ssum)
        aggb = pieces[0] if npass == 1 else jnp.concatenate(pieces, axis=1)
        cntb = cnt_refs[0][0][:, 0:1]
        for rr in cnt_refs[1:]:
            cntb = cntb + rr[0][:, 0:1]
        deg = jnp.maximum(cntb, 1.0)
        x = jnp.dot(aggb, Wr[...], preferred_element_type=jnp.float32) / deg
        x = x + br[...] + hp[...]
        mu = jnp.mean(x, axis=1, keepdims=True)
        var = jnp.mean((x - mu) * (x - mu), axis=1, keepdims=True)
        y = (x - mu) * lax.rsqrt(var + 1e-5) * gr[...] + betar[...]
        out[...] = jnp.maximum(y, 0.0)

    return pl.pallas_call(
        body,
        grid=(grid,),
        in_specs=in_specs,
        out_specs=pl.BlockSpec((blk, d), lambda i: (i, 0)),
        out_shape=jax.ShapeDtypeStruct((n, d), jnp.float32),
    )(*operands)


def _stage(h_src, h_prev, src, dst, w, Wm, b, g, beta, npass, blk):
    n_src, d = h_src.shape
    n_dst = h_prev.shape[0]
    width = d // npass
    super_ = 8 if width == 16 else 4
    step = super_ * CHUNK
    e_pad = _ceil_to(src.shape[0], NW * step)
    if (e_pad // (NW * step)) % 2:
        e_pad += NW * step
    sp, dp, wp = _pad_edges(src, dst, n_dst, e_pad, w)
    hv = h_src.reshape(n_src * npass, width)
    agg, cnt = _seg_sum_call(hv, sp, dp, wp, n_dst, e_pad, npass, width,
                             merge_cnt=(width > 16), super_=super_)
    return _tc_dense(h_prev, Wm, b, g, beta, agg, cnt, npass, blk)


def kernel(h_pin, h_net, overlap_weights, W_p2n, b_p2n, W_n2n, b_n2n, W_n2p,
           b_n2p, ln1_g, ln1_b, ln2_g, ln2_b, lnp_g, lnp_b,
           src_p2n, dst_p2n, src_n2n, dst_n2n, src_n2p, dst_n2p):
    h_net1 = _stage(h_pin, h_net, src_p2n, dst_p2n, None,
                    W_p2n, b_p2n, ln1_g, ln1_b, npass=2, blk=1000)
    h_net2 = _stage(h_net1, h_net1, src_n2n, dst_n2n, overlap_weights,
                    W_n2n, b_n2n, ln2_g, ln2_b, npass=2, blk=1000)
    h_pin_out = _stage(h_net2, h_pin, src_n2p, dst_n2p, None,
                       W_n2p, b_n2p, lnp_g, lnp_b, npass=8, blk=1000)
    return (h_pin_out, h_net2)


# speedup vs baseline: 2.3343x; 1.0550x over previous
import jax
import jax.numpy as jnp
from jax import lax
from jax.experimental import pallas as pl
from jax.experimental.pallas import tpu as pltpu
from jax.experimental.pallas import tpu_sc as plsc

NC, NS, LANES = 2, 16, 16
NW = NC * NS
CHUNK = 128


def _ceil_to(x, m):
    return -(-x // m) * m


def _pad_edges(src, dst, n_dst, e_pad, w=None):
    e = src.shape[0]
    pad = e_pad - e
    src_p = jnp.concatenate([src, jnp.zeros((pad,), jnp.int32)])
    dst_p = jnp.concatenate([dst, jnp.full((pad,), n_dst, jnp.int32)])
    if w is None:
        return src_p, dst_p, None
    w_p = jnp.concatenate([w, jnp.zeros((pad,), jnp.float32)])
    return src_p, dst_p, w_p


def _stripe_rows(n_rows):
    per = -(-n_rows // NS)
    return _ceil_to(per, 32)


def _seg_sum_call(hv, src_p, dst_p, w_p, n_dst, npass, width,
                  merge_cnt, super_, u0, u1):
    SUPER = super_
    STEP = SUPER * CHUNK
    stripe = _stripe_rows(n_dst + 1)
    acc_rows = stripe * NS
    assert u0 >= 1 and u1 >= 1
    weighted = w_p is not None
    if not merge_cnt:
        assert width == 16

    zeros = jnp.zeros((stripe, width), jnp.float32)
    zeros16 = jnp.zeros((stripe, 16), jnp.float32)
    ones = jnp.ones((CHUNK, 16), jnp.float32)

    mesh = plsc.VectorSubcoreMesh(core_axis_name="c", subcore_axis_name="s")
    scratch = [
        pltpu.VMEM_SHARED((acc_rows, width), jnp.float32),
        pltpu.VMEM((2, STEP, width), jnp.float32),
        pltpu.VMEM((2, SUPER, CHUNK), jnp.int32),
        pltpu.VMEM((2, SUPER, CHUNK), jnp.int32),
        pltpu.VMEM((2, SUPER, CHUNK), jnp.int32),
        pltpu.VMEM((2, SUPER, CHUNK), jnp.int32),
        pltpu.VMEM((CHUNK, 16), jnp.float32),
        pltpu.SemaphoreType.DMA,
        pltpu.SemaphoreType.DMA,
        pltpu.SemaphoreType.DMA,
        pltpu.SemaphoreType.DMA,
        pltpu.SemaphoreType.DMA,
        pltpu.SemaphoreType.DMA,
    ]
    if merge_cnt:
        scratch.append(pltpu.VMEM_SHARED((acc_rows, 16), jnp.float32))
    if weighted:
        scratch.append(pltpu.VMEM((2, SUPER, CHUNK), jnp.float32))
    out_shapes = (jax.ShapeDtypeStruct((npass, NC, acc_rows, width),
                                       jnp.float32),
                  jax.ShapeDtypeStruct((NC, acc_rows, 16), jnp.float32))

    def body(*refs):
        it = iter(refs)
        h_r = next(it)
        src_r = next(it)
        dst_r = next(it)
        w_r = next(it) if weighted else None
        z_r = next(it)
        z16_r = next(it)
        ones_r = next(it)
        agg_o = next(it)
        cnt_o = next(it)
        acc = next(it)
        rows = next(it)
        idx_s = next(it)
        idx_d = next(it)
        idxd2 = next(it)
        idxp = next(it)
        ones_v = next(it)
        semI = (next(it), next(it))
        semG = (next(it), next(it))
        semS = (next(it), next(it))
        cacc = next(it) if merge_cnt else None
        w_v = next(it) if weighted else None

        c = lax.axis_index("c")
        s = lax.axis_index("s")
        e0 = u0 * NS * STEP * 2
        epw_c = jnp.where(c == 0, u0, u1) * (2 * STEP)
        off_w = c * e0 + s * epw_c
        nsteps = jnp.where(c == 0, 2 * u0, 2 * u1)
        npairs = jnp.where(c == 0, u0, u1)
        q = stripe // 4
        pltpu.sync_copy(ones_r, ones_v)

        total_passes = npass if merge_cnt else npass + 1
        for f in range(total_passes):
            gather_pass = f < npass
            do_cnt = (merge_cnt and f == 0) or not gather_pass
            cnt_acc = cacc if merge_cnt else acc

            for j in range(4):
                pltpu.sync_copy(z_r.at[pl.ds(j * q, q)],
                                acc.at[pl.ds(s * stripe + j * q, q)])
            if merge_cnt and f == 0:
                pltpu.sync_copy(z16_r, cacc.at[pl.ds(s * stripe, stripe)])
            plsc.subcore_barrier()

            def fire_idx(par, i):
                base = off_w + i * STEP
                for j in range(SUPER):
                    bj = base + j * CHUNK
                    pltpu.async_copy(dst_r.at[pl.ds(bj, CHUNK)],
                                     idx_d.at[par].at[j], semI[par])
                    if gather_pass:
                        pltpu.async_copy(src_r.at[pl.ds(bj, CHUNK)],
                                         idx_s.at[par].at[j], semI[par])
                        if weighted:
                            pltpu.async_copy(w_r.at[pl.ds(bj, CHUNK)],
                                             w_v.at[par].at[j], semI[par])

            def wait_idx(par):
                for j in range(SUPER):
                    pltpu.make_async_copy(dst_r.at[pl.ds(0, CHUNK)],
                                          idx_d.at[par].at[j],
                                          semI[par]).wait()
                    if gather_pass:
                        pltpu.make_async_copy(src_r.at[pl.ds(0, CHUNK)],
                                              idx_s.at[par].at[j],
                                              semI[par]).wait()
                        if weighted:
                            pltpu.make_async_copy(w_r.at[pl.ds(0, CHUNK)],
                                                  w_v.at[par].at[j],
                                                  semI[par]).wait()

            def mk_idx(par, f=f):
                for j in range(SUPER):
                    def mk(k, carry2, j=j):
                        sl = pl.ds(k * LANES, LANES)
                        idxp[par, j, sl] = idx_s[par, j, sl] * npass + f
                        return carry2
                    lax.fori_loop(0, CHUNK // LANES, mk, 0)

            def fire_gather(par):
                for j in range(SUPER):
                    pltpu.async_copy(h_r.at[idxp.at[par].at[j]],
                                     rows.at[par].at[pl.ds(j * CHUNK, CHUNK)],
                                     semG[par])

            def wait_gather(par):
                for j in range(SUPER):
                    pltpu.make_async_copy(
                        h_r.at[pl.ds(0, CHUNK)],
                        rows.at[par].at[pl.ds(j * CHUNK, CHUNK)],
                        semG[par]).wait()

            def scale_rows(par):
                for j in range(SUPER):
                    def scale(k, carry2, j=j):
                        wv16 = w_v[par, j, pl.ds(k * LANES, LANES)]
                        for u in range(LANES):
                            r = j * CHUNK + k * LANES + u
                            wvu = wv16[u]
                            for t in range(width // LANES):
                                sl = pl.ds(t * LANES, LANES)
                                rows[par, r, sl] = rows[par, r, sl] * wvu
                        return carry2
                    lax.fori_loop(0, CHUNK // LANES, scale, 0)

            def cp_idxd(par):
                for j in range(SUPER):
                    def cp(k, carry2, j=j):
                        sl = pl.ds(k * LANES, LANES)
                        idxd2[par, j, sl] = idx_d[par, j, sl]
                        return carry2
                    lax.fori_loop(0, CHUNK // LANES, cp, 0)

            def fire_scatters(par):
                for j in range(SUPER):
                    if gather_pass:
                        pltpu.async_copy(
                            rows.at[par].at[pl.ds(j * CHUNK, CHUNK)],
                            acc.at[idxd2.at[par].at[j]], semS[par], add=True)
                    if do_cnt:
                        pltpu.async_copy(ones_v,
                                         cnt_acc.at[idxd2.at[par].at[j]],
                                         semS[par], add=True)

            def drain_scatters(par):
                for j in range(SUPER):
                    if gather_pass:
                        pltpu.make_async_copy(
                            rows.at[par].at[pl.ds(j * CHUNK, CHUNK)],
                            acc.at[pl.ds(0, CHUNK)], semS[par]).wait()
                    if do_cnt:
                        pltpu.make_async_copy(
                            ones_v, cnt_acc.at[pl.ds(0, CHUNK)],
                            semS[par]).wait()

            fire_idx(0, 0)
            fire_idx(1, 1)
            wait_idx(0)
            cp_idxd(0)
            if gather_pass:
                mk_idx(0)
                fire_gather(0)

            def pair(g, carry):
                for par in (0, 1):
                    i = g * 2 + par

                    if gather_pass:
                        wait_gather(par)
                        if weighted:
                            scale_rows(par)
                    fire_scatters(par)

                    @pl.when(i + 2 < nsteps)
                    def _():
                        fire_idx(par, i + 2)

                    @pl.when(i >= 1)
                    def _():
                        drain_scatters(1 - par)

                    @pl.when(i + 1 < nsteps)
                    def _():
                        wait_idx(1 - par)
                        cp_idxd(1 - par)
                        if gather_pass:
                            mk_idx(1 - par)
                            fire_gather(1 - par)
                return carry

            lax.fori_loop(0, npairs, pair, 0)
            drain_scatters(1)
            plsc.subcore_barrier()
            if gather_pass:
                for j in range(4):
                    slj = pl.ds(s * stripe + j * q, q)
                    pltpu.sync_copy(acc.at[slj], agg_o.at[f].at[c].at[slj])
            if do_cnt:
                for j in range(4):
                    slj = pl.ds(s * stripe + j * q, q)
                    pltpu.sync_copy(cnt_acc.at[slj], cnt_o.at[c].at[slj])
            plsc.subcore_barrier()

    fn = pl.kernel(body, out_shapes, mesh=mesh, scratch_types=scratch,
                   compiler_params=pltpu.CompilerParams(
                       use_tc_tiling_on_sc=False))
    if weighted:
        return fn(hv, src_p, dst_p, w_p, zeros, zeros16, ones)
    return fn(hv, src_p, dst_p, zeros, zeros16, ones)


def _tc_dense(h_prev, W, b, g, beta, agg, cnt, npass, blk):
    n, d = h_prev.shape
    width = agg.shape[3]
    grid = n // blk
    b2 = b.reshape(1, d)
    g2 = g.reshape(1, d)
    beta2 = beta.reshape(1, d)

    in_specs = [
        pl.BlockSpec((blk, d), lambda i: (i, 0)),
        pl.BlockSpec((d, d), lambda i: (0, 0)),
        pl.BlockSpec((1, d), lambda i: (0, 0)),
        pl.BlockSpec((1, d), lambda i: (0, 0)),
        pl.BlockSpec((1, d), lambda i: (0, 0)),
    ]
    operands = [h_prev, W, b2, g2, beta2]
    for f in range(npass):
        for cc in range(NC):
            in_specs.append(pl.BlockSpec(
                (1, 1, blk, width), lambda i, f=f, cc=cc: (f, cc, i, 0)))
            operands.append(agg)
    for cc in range(NC):
        in_specs.append(pl.BlockSpec(
            (1, blk, 16), lambda i, cc=cc: (cc, i, 0)))
        operands.append(cnt)

    def body(hp, Wr, br, gr, betar, *rest):
        out = rest[-1]
        agg_refs = rest[:npass * NC]
        cnt_refs = rest[npass * NC:-1]
        pieces = []
        for f in range(npass):
            ssum = agg_refs[f * NC][0, 0]
            for cc in range(1, NC):
                ssum = ssum + agg_refs[f * NC + cc][0, 0]
            pieces.append(ssum)
        aggb = pieces[0] if npass == 1 else jnp.concatenate(pieces, axis=1)
        cntb = cnt_refs[0][0][:, 0:1]
        for rr in cnt_refs[1:]:
            cntb = cntb + rr[0][:, 0:1]
        deg = jnp.maximum(cntb, 1.0)
        x = jnp.dot(aggb, Wr[...], preferred_element_type=jnp.float32) / deg
        x = x + br[...] + hp[...]
        mu = jnp.mean(x, axis=1, keepdims=True)
        var = jnp.mean((x - mu) * (x - mu), axis=1, keepdims=True)
        y = (x - mu) * lax.rsqrt(var + 1e-5) * gr[...] + betar[...]
        out[...] = jnp.maximum(y, 0.0)

    return pl.pallas_call(
        body,
        grid=(grid,),
        in_specs=in_specs,
        out_specs=pl.BlockSpec((blk, d), lambda i: (i, 0)),
        out_shape=jax.ShapeDtypeStruct((n, d), jnp.float32),
    )(*operands)


def _stage(h_src, h_prev, src, dst, w, Wm, b, g, beta, npass, blk):
    n_src, d = h_src.shape
    n_dst = h_prev.shape[0]
    width = d // npass
    super_ = 4
    step = super_ * CHUNK
    unit = NS * step * 2
    units = -(-src.shape[0] // unit)
    u1 = max(1, round(units * 0.31))
    u0 = units - u1
    e_pad = units * unit
    sp, dp, wp = _pad_edges(src, dst, n_dst, e_pad, w)
    hv = h_src.reshape(n_src * npass, width)
    agg, cnt = _seg_sum_call(hv, sp, dp, wp, n_dst, npass, width,
                             merge_cnt=(width > 16), super_=super_,
                             u0=u0, u1=u1)
    return _tc_dense(h_prev, Wm, b, g, beta, agg, cnt, npass, blk)


def kernel(h_pin, h_net, overlap_weights, W_p2n, b_p2n, W_n2n, b_n2n, W_n2p,
           b_n2p, ln1_g, ln1_b, ln2_g, ln2_b, lnp_g, lnp_b,
           src_p2n, dst_p2n, src_n2n, dst_n2n, src_n2p, dst_n2p):
    h_net1 = _stage(h_pin, h_net, src_p2n, dst_p2n, None,
                    W_p2n, b_p2n, ln1_g, ln1_b, npass=2, blk=1000)
    h_net2 = _stage(h_net1, h_net1, src_n2n, dst_n2n, overlap_weights,
                    W_n2n, b_n2n, ln2_g, ln2_b, npass=2, blk=1000)
    h_pin_out = _stage(h_net2, h_pin, src_n2p, dst_n2p, None,
                       W_n2p, b_n2p, lnp_g, lnp_b, npass=8, blk=1000)
    return (h_pin_out, h_net2)
